# baseline (speedup 1.0000x reference)
import functools

import jax
import jax.numpy as jnp
from jax import lax
from jax.experimental import pallas as pl
from jax.experimental.pallas import tpu as pltpu
from jax.experimental.pallas import tpu_sc as plsc

N, C, D, H, W = 4, 96, 8, 28, 28
Do, Ho, Wo = 16, 56, 56
IN_ROW = D * H * W
OUT_ROW = Do * Ho * Wo
ROWS = N * C
L = 16
CP = 128
N_STRIDE = Do * Ho * Wo * CP
OUT_PAD = N * N_STRIDE

_info = plsc.get_sparse_core_info()
NUM_CORES = _info.num_cores
NUM_SUBCORES = _info.num_subcores
ROWS_PER_W = ROWS // (NUM_CORES * NUM_SUBCORES)

ZCHUNK = 16384
ZPER_W = OUT_PAD // (NUM_CORES * NUM_SUBCORES)

_mesh = plsc.VectorSubcoreMesh(core_axis_name="c", subcore_axis_name="s")


@functools.partial(
    pl.kernel,
    out_type=jax.ShapeDtypeStruct((OUT_PAD,), jnp.float32),
    mesh=_mesh,
    compiler_params=pltpu.CompilerParams(needs_layout_passes=False),
    scratch_types=[
        pltpu.VMEM((IN_ROW + L,), jnp.int32),
        pltpu.VMEM((IN_ROW,), jnp.float32),
        pltpu.VMEM((IN_ROW,), jnp.int32),
        pltpu.VMEM((ZCHUNK,), jnp.float32),
    ],
)
def _unpool_sc(key_hbm, val_hbm, out_hbm, key_v, val_v, pos_v, zbuf):
    cid = lax.axis_index("c")
    sid = lax.axis_index("s")

    zeros = jnp.zeros((L,), jnp.float32)

    def zb_body(i, c):
        base = i * (8 * L)
        for u in range(8):
            zbuf[pl.ds(base + u * L, L)] = zeros
        return c

    lax.fori_loop(0, ZCHUNK // (8 * L), zb_body, 0, unroll=False)

    zbase = (cid * NUM_SUBCORES + sid) * ZPER_W

    def zdma_body(i, c):
        pltpu.sync_copy(zbuf, out_hbm.at[pl.ds(zbase + i * ZCHUNK, ZCHUNK)])
        return c

    lax.fori_loop(0, ZPER_W // ZCHUNK, zdma_body, 0, unroll=False)

    plsc.subcore_barrier()

    key_v[pl.ds(IN_ROW, L)] = jnp.full((L,), -1, jnp.int32)

    def row_body(r, carry):
        row = cid * (NUM_SUBCORES * ROWS_PER_W) + sid * ROWS_PER_W + r
        pltpu.sync_copy(key_hbm.at[pl.ds(row * IN_ROW, IN_ROW)],
                        key_v.at[pl.ds(0, IN_ROW)])
        pltpu.sync_copy(val_hbm.at[pl.ds(row * IN_ROW, IN_ROW)], val_v)

        n = row // C
        ch = row % C
        row_base = row * OUT_ROW
        nbase = n * N_STRIDE
        trash = nbase + C

        def pos_body(i, c):
            for u in range(4):
                b = (i * 4 + u) * L
                k = key_v[pl.ds(b, L)]
                kn = key_v[pl.ds(b + 1, L)]
                keep = k != kn
                lk = k - row_base
                p = nbase + lk * CP + ch
                pos_v[pl.ds(b, L)] = jnp.where(keep, p, trash)
            return c

        lax.fori_loop(0, IN_ROW // (4 * L), pos_body, 0, unroll=False)

        pltpu.sync_copy(val_v, out_hbm.at[pos_v])
        return carry

    lax.fori_loop(0, ROWS_PER_W, row_body, 0, unroll=False)


def kernel(x, indices, output_size, stride, padding):
    xf = x.reshape(-1)
    rowid = jnp.arange(ROWS, dtype=jnp.int32) * OUT_ROW
    keys = (indices.reshape(ROWS, IN_ROW) + rowid[:, None]).reshape(-1)
    skeys, svals = lax.sort((keys, xf), num_keys=1, is_stable=False)
    flat = _unpool_sc(skeys, svals)
    out = flat.reshape(N, Do, Ho, Wo, CP)[..., :C]
    return jnp.transpose(out, (0, 4, 1, 2, 3))


# speedup vs baseline: 2.7440x; 2.7440x over previous
import functools

import jax
import jax.numpy as jnp
from jax import lax
from jax.experimental import pallas as pl
from jax.experimental.pallas import tpu as pltpu
from jax.experimental.pallas import tpu_sc as plsc

N, C, D, H, W = 4, 96, 8, 28, 28
Do, Ho, Wo = 16, 56, 56
IN_ROW = D * H * W
OUT_ROW = Do * Ho * Wo
ROWS = N * C
L = 16

_info = plsc.get_sparse_core_info()
NUM_CORES = _info.num_cores
NUM_SUBCORES = _info.num_subcores
NW = NUM_CORES * NUM_SUBCORES
ROWS_PER_W = ROWS // NW

_mesh = plsc.VectorSubcoreMesh(core_axis_name="c", subcore_axis_name="s")


@functools.partial(
    pl.kernel,
    out_type=jax.ShapeDtypeStruct((N, C, Do, Ho, Wo), jnp.float32),
    mesh=_mesh,
    compiler_params=pltpu.CompilerParams(needs_layout_passes=False),
    scratch_types=[
        pltpu.VMEM((IN_ROW + L,), jnp.int32),
        pltpu.VMEM((IN_ROW,), jnp.float32),
        pltpu.VMEM((Do, Ho, Wo), jnp.float32),
    ],
)
def _unpool_sc(key_hbm, val_hbm, out_hbm, key_v, val_v, vol_v):
    cid = lax.axis_index("c")
    sid = lax.axis_index("s")
    wid = sid * NUM_CORES + cid

    zeros = jnp.zeros((L,), jnp.float32)

    key_v[pl.ds(IN_ROW, L)] = jnp.full((L,), -1, jnp.int32)

    def row_body(r, carry):
        row = wid * ROWS_PER_W + r
        pltpu.sync_copy(key_hbm.at[pl.ds(row * IN_ROW, IN_ROW)],
                        key_v.at[pl.ds(0, IN_ROW)])
        pltpu.sync_copy(val_hbm.at[pl.ds(row * IN_ROW, IN_ROW)], val_v)

        def zero_body(i, c):
            d = i // Ho
            h = i % Ho
            for wv in (0, L, 2 * L, Wo - L):
                vol_v[d, h, pl.ds(wv, L)] = zeros
            return c

        lax.fori_loop(0, Do * Ho, zero_body, 0, unroll=False)

        row_base = row * OUT_ROW
        n = row // C
        ch = row % C

        def scat_body(i, c):
            for u in range(2):
                b = (i * 2 + u) * L
                k = key_v[pl.ds(b, L)]
                kn = key_v[pl.ds(b + 1, L)]
                keep = k != kn
                lk = k - row_base
                id_ = lk // (Ho * Wo)
                r1 = lk - id_ * (Ho * Wo)
                ih = r1 // Wo
                iw = r1 - ih * Wo
                vv = val_v[pl.ds(b, L)]
                plsc.store_scatter(vol_v, [id_, ih, iw], vv, mask=keep)
            return c

        lax.fori_loop(0, IN_ROW // (2 * L), scat_body, 0, unroll=False)

        pltpu.sync_copy(vol_v, out_hbm.at[n, ch])
        return carry

    lax.fori_loop(0, ROWS_PER_W, row_body, 0, unroll=False)


def kernel(x, indices, output_size, stride, padding):
    xf = x.reshape(-1)
    rowid = jnp.arange(ROWS, dtype=jnp.int32) * OUT_ROW
    keys = (indices.reshape(ROWS, IN_ROW) + rowid[:, None]).reshape(-1)
    skeys, svals = lax.sort((keys, xf), num_keys=1, is_stable=False)
    return _unpool_sc(skeys, svals)


# speedup vs baseline: 2.8545x; 1.0403x over previous
import functools

import jax
import jax.numpy as jnp
from jax import lax
from jax.experimental import pallas as pl
from jax.experimental.pallas import tpu as pltpu
from jax.experimental.pallas import tpu_sc as plsc

N, C, D, H, W = 4, 96, 8, 28, 28
Do, Ho, Wo = 16, 56, 56
IN_ROW = D * H * W
OUT_ROW = Do * Ho * Wo
ROWS = N * C
L = 16

_info = plsc.get_sparse_core_info()
NUM_CORES = _info.num_cores
NUM_SUBCORES = _info.num_subcores
NW = NUM_CORES * NUM_SUBCORES
ROWS_PER_W = ROWS // NW

_mesh = plsc.VectorSubcoreMesh(core_axis_name="c", subcore_axis_name="s")


@functools.partial(
    pl.kernel,
    out_type=jax.ShapeDtypeStruct((ROWS, OUT_ROW), jnp.float32),
    mesh=_mesh,
    compiler_params=pltpu.CompilerParams(needs_layout_passes=False),
    scratch_types=[
        pltpu.VMEM((IN_ROW + L,), jnp.int32),
        pltpu.VMEM((IN_ROW + L,), jnp.int32),
        pltpu.VMEM((IN_ROW,), jnp.float32),
        pltpu.VMEM((OUT_ROW,), jnp.float32),
        pltpu.VMEM((OUT_ROW,), jnp.float32),
        pltpu.SemaphoreType.DMA,
        pltpu.SemaphoreType.DMA,
    ],
)
def _unpool_sc(key_hbm, val_hbm, out_hbm, key_v0, key_v1, val_v,
               row_v0, row_v1, sem0, sem1):
    cid = lax.axis_index("c")
    sid = lax.axis_index("s")
    wid = sid * NUM_CORES + cid

    zeros = jnp.zeros((L,), jnp.float32)
    sems = (sem0, sem1)
    keys = (key_v0, key_v1)
    rows = (row_v0, row_v1)

    key_v0[pl.ds(IN_ROW, L)] = jnp.full((L,), -1, jnp.int32)
    key_v1[pl.ds(IN_ROW, L)] = jnp.full((L,), -1, jnp.int32)

    def zero_body(i, c):
        base = i * (8 * L)
        for rv in rows:
            for u in range(8):
                rv[pl.ds(base + u * L, L)] = zeros
        return c

    lax.fori_loop(0, OUT_ROW // (8 * L), zero_body, 0, unroll=False)

    for r in range(ROWS_PER_W):
        b = r % 2
        row = wid * ROWS_PER_W + r
        row_base = row * OUT_ROW

        if r >= 2:
            old_base = (row - 2) * OUT_ROW
            pltpu.make_async_copy(
                rows[b], out_hbm.at[row - 2], sems[b]
            ).wait()

            def rezero_body(i, c, _kv=keys[b], _rv=rows[b], _ob=old_base):
                for u in range(4):
                    o = (i * 4 + u) * L
                    lk = _kv[pl.ds(o, L)] - _ob
                    plsc.store_scatter(_rv, [lk], zeros)
                return c

            lax.fori_loop(0, IN_ROW // (4 * L), rezero_body, 0, unroll=False)

        pltpu.sync_copy(key_hbm.at[pl.ds(row * IN_ROW, IN_ROW)],
                        keys[b].at[pl.ds(0, IN_ROW)])
        pltpu.sync_copy(val_hbm.at[pl.ds(row * IN_ROW, IN_ROW)], val_v)

        def scat_body(i, c, _kv=keys[b], _rv=rows[b], _rb=row_base):
            for u in range(4):
                o = (i * 4 + u) * L
                k = _kv[pl.ds(o, L)]
                kn = _kv[pl.ds(o + 1, L)]
                keep = k != kn
                lk = k - _rb
                vv = val_v[pl.ds(o, L)]
                plsc.store_scatter(_rv, [lk], vv, mask=keep)
            return c

        lax.fori_loop(0, IN_ROW // (4 * L), scat_body, 0, unroll=False)

        pltpu.async_copy(rows[b], out_hbm.at[row], sems[b])

    for r in (ROWS_PER_W - 2, ROWS_PER_W - 1):
        b = r % 2
        row = wid * ROWS_PER_W + r
        pltpu.make_async_copy(rows[b], out_hbm.at[row], sems[b]).wait()


def kernel(x, indices, output_size, stride, padding):
    xf = x.reshape(-1)
    rowid = jnp.arange(ROWS, dtype=jnp.int32) * OUT_ROW
    keys = (indices.reshape(ROWS, IN_ROW) + rowid[:, None]).reshape(-1)
    skeys, svals = lax.sort((keys, xf), num_keys=1, is_stable=False)
    out = _unpool_sc(skeys, svals)
    return out.reshape(N, C, Do, Ho, Wo)
